# baseline (speedup 1.0000x reference)
import functools

import jax
import jax.numpy as jnp
from jax import lax
from jax.experimental import pallas as pl
from jax.experimental.pallas import tpu as pltpu
from jax.experimental.pallas import tpu_sc as plsc

BATCH = 64
SEQ = 2048
D_IN = 1024
D_PROJ = 256

_NUM_W = 32
_ROWS_PER_W = 4


def _gather_body(idx_hbm, table_hbm, out_hbm, idx_v, rows_v, sem):
    wid = lax.axis_index("s") * 2 + lax.axis_index("c")

    pltpu.sync_copy(idx_hbm.at[wid], idx_v)
    pltpu.async_copy(table_hbm.at[idx_v], rows_v, sem).wait()
    pltpu.sync_copy(rows_v, out_hbm.at[pl.ds(wid * _ROWS_PER_W, _ROWS_PER_W)])


_gather_rows = functools.partial(
    pl.kernel,
    mesh=plsc.VectorSubcoreMesh(core_axis_name="c", subcore_axis_name="s"),
    out_type=jax.ShapeDtypeStruct((2 * BATCH, D_IN), jnp.float32),
    scratch_types=[
        pltpu.VMEM((_ROWS_PER_W,), jnp.int32),
        pltpu.VMEM((_ROWS_PER_W, D_IN), jnp.float32),
        pltpu.SemaphoreType.DMA,
    ],
)(_gather_body)


def _proj_body(g_ref, w_ref, b_ref, o_ref):
    r = lax.dot_general(
        g_ref[...], w_ref[...],
        dimension_numbers=(((1,), (1,)), ((), ())),
        preferred_element_type=jnp.float32,
    )
    r = r + b_ref[...]
    o_ref[:, :D_PROJ] = r[:BATCH, :]
    o_ref[:, D_PROJ:] = r[BATCH:, :]


def kernel(encoded_input, start_ids, end_ids, W, b):
    table = encoded_input.reshape(BATCH * SEQ, D_IN)
    offs = jnp.arange(BATCH, dtype=jnp.int32) * SEQ
    idx = jnp.concatenate(
        [start_ids.astype(jnp.int32) + offs, end_ids.astype(jnp.int32) + offs]
    ).reshape(_NUM_W, _ROWS_PER_W)
    gathered = _gather_rows(idx, table)
    return pl.pallas_call(
        _proj_body,
        out_shape=jax.ShapeDtypeStruct((BATCH, 2 * D_PROJ), jnp.float32),
    )(gathered, W, b.reshape(1, D_PROJ))


# speedup vs baseline: 1.0648x; 1.0648x over previous
import functools

import jax
import jax.numpy as jnp
from jax import lax
from jax.experimental import pallas as pl
from jax.experimental.pallas import tpu as pltpu
from jax.experimental.pallas import tpu_sc as plsc

BATCH = 64
SEQ = 2048
D_IN = 1024
D_PROJ = 256

_NUM_W = 16
_ROWS_PER_W = 8
_CHUNK = 4


def _gather_body(idx_hbm, table_hbm, out_hbm, idx_a, idx_b, rows_a, rows_b,
                 sem_i, sem_g, sem_o):
    wid = lax.axis_index("s")
    base = wid * _ROWS_PER_W

    ia = pltpu.async_copy(idx_hbm.at[2 * wid], idx_a, sem_i)
    ib = pltpu.async_copy(idx_hbm.at[2 * wid + 1], idx_b, sem_i)
    ia.wait()
    ga = pltpu.async_copy(table_hbm.at[idx_a], rows_a, sem_g)
    ib.wait()
    gb = pltpu.async_copy(table_hbm.at[idx_b], rows_b, sem_g)
    ga.wait()
    oa = pltpu.async_copy(rows_a, out_hbm.at[pl.ds(base, _CHUNK)], sem_o)
    gb.wait()
    ob = pltpu.async_copy(rows_b, out_hbm.at[pl.ds(base + _CHUNK, _CHUNK)], sem_o)
    oa.wait()
    ob.wait()


_gather_rows = functools.partial(
    pl.kernel,
    mesh=plsc.VectorSubcoreMesh(core_axis_name="c", subcore_axis_name="s", num_cores=1),
    out_type=jax.ShapeDtypeStruct((2 * BATCH, D_IN), jnp.float32),
    scratch_types=[
        pltpu.VMEM((_CHUNK,), jnp.int32),
        pltpu.VMEM((_CHUNK,), jnp.int32),
        pltpu.VMEM((_CHUNK, D_IN), jnp.float32),
        pltpu.VMEM((_CHUNK, D_IN), jnp.float32),
        pltpu.SemaphoreType.DMA,
        pltpu.SemaphoreType.DMA,
        pltpu.SemaphoreType.DMA,
    ],
)(_gather_body)


def _proj_body(g_ref, w_ref, b_ref, o_ref):
    r = lax.dot_general(
        g_ref[...], w_ref[...],
        dimension_numbers=(((1,), (1,)), ((), ())),
        preferred_element_type=jnp.float32,
    )
    r = r + b_ref[...]
    o_ref[:, :D_PROJ] = r[:BATCH, :]
    o_ref[:, D_PROJ:] = r[BATCH:, :]


def kernel(encoded_input, start_ids, end_ids, W, b):
    table = encoded_input.reshape(BATCH * SEQ, D_IN)
    offs = jnp.arange(BATCH, dtype=jnp.int32) * SEQ
    idx = jnp.concatenate(
        [start_ids.astype(jnp.int32) + offs, end_ids.astype(jnp.int32) + offs]
    ).reshape(2 * _NUM_W, _CHUNK)
    gathered = _gather_rows(idx, table)
    return pl.pallas_call(
        _proj_body,
        out_shape=jax.ShapeDtypeStruct((BATCH, 2 * D_PROJ), jnp.float32),
    )(gathered, W, b.reshape(1, D_PROJ))


# speedup vs baseline: 1.0650x; 1.0002x over previous
import functools

import jax
import jax.numpy as jnp
from jax import lax
from jax.experimental import pallas as pl
from jax.experimental.pallas import tpu as pltpu
from jax.experimental.pallas import tpu_sc as plsc

BATCH = 64
SEQ = 2048
D_IN = 1024
D_PROJ = 256

_NUM_W = 16
_ROWS_PER_W = 8
_CHUNK = 4


def _gather_body(start_hbm, end_hbm, table_hbm, out_hbm, ids_v, idx_v, rows_v, sem):
    wid = lax.axis_index("s")
    b0 = (wid & 7) * _ROWS_PER_W

    @pl.when(wid < 8)
    def _():
        pltpu.sync_copy(start_hbm.at[pl.ds(b0, _ROWS_PER_W)],
                        ids_v.at[pl.ds(0, _ROWS_PER_W)])

    @pl.when(wid >= 8)
    def _():
        pltpu.sync_copy(end_hbm.at[pl.ds(b0, _ROWS_PER_W)],
                        ids_v.at[pl.ds(0, _ROWS_PER_W)])

    idx_v[...] = ids_v[...] + (b0 + lax.iota(jnp.int32, 16)) * SEQ

    pltpu.async_copy(table_hbm.at[idx_v.at[pl.ds(0, _ROWS_PER_W)]], rows_v,
                     sem).wait()
    pltpu.sync_copy(rows_v, out_hbm.at[pl.ds(wid * _ROWS_PER_W, _ROWS_PER_W)])


_gather_rows = functools.partial(
    pl.kernel,
    mesh=plsc.VectorSubcoreMesh(core_axis_name="c", subcore_axis_name="s", num_cores=1),
    out_type=jax.ShapeDtypeStruct((2 * BATCH, D_IN), jnp.float32),
    scratch_types=[
        pltpu.VMEM((16,), jnp.int32),
        pltpu.VMEM((16,), jnp.int32),
        pltpu.VMEM((_ROWS_PER_W, D_IN), jnp.float32),
        pltpu.SemaphoreType.DMA,
    ],
)(_gather_body)


def _proj_body(g_ref, w_ref, b_ref, o_ref):
    r = lax.dot_general(
        g_ref[...], w_ref[...],
        dimension_numbers=(((1,), (1,)), ((), ())),
        preferred_element_type=jnp.float32,
    )
    r = r + b_ref[...]
    o_ref[:, :D_PROJ] = r[:BATCH, :]
    o_ref[:, D_PROJ:] = r[BATCH:, :]


def kernel(encoded_input, start_ids, end_ids, W, b):
    table = encoded_input.reshape(BATCH * SEQ, D_IN)
    gathered = _gather_rows(
        start_ids.astype(jnp.int32), end_ids.astype(jnp.int32), table
    )
    return pl.pallas_call(
        _proj_body,
        out_shape=jax.ShapeDtypeStruct((BATCH, 2 * D_PROJ), jnp.float32),
    )(gathered, W, b.reshape(1, D_PROJ))
